# baseline (speedup 1.0000x reference)
import functools

import jax
import jax.numpy as jnp
from jax import lax
from jax.experimental import pallas as pl
from jax.experimental.pallas import tpu as pltpu
from jax.experimental.pallas import tpu_sc as plsc

N = 10000
D = 128
E = 320000

NC = 2
NS = 16
NW = NC * NS

E_TILE = E // NW
CHUNK = 80
N_CHUNKS = E_TILE // CHUNK
ROWS_PER_TILE = 624
TAIL_ROW = NS * ROWS_PER_TILE
TAIL = N - TAIL_ROW
ZROWS = 16

BLOCK_M = 1000


def _mm_body(x_ref, w_ref, b_ref, o_ref, *, relu):
    acc = jnp.dot(x_ref[...], w_ref[...], preferred_element_type=jnp.float32)
    acc = acc + b_ref[...]
    if relu:
        acc = jnp.maximum(acc, 0.0)
    o_ref[...] = acc


def _mm(x, wt, b, relu=False):
    grid = (N // BLOCK_M,)
    return pl.pallas_call(
        functools.partial(_mm_body, relu=relu),
        grid=grid,
        in_specs=[
            pl.BlockSpec((BLOCK_M, D), lambda i: (i, 0)),
            pl.BlockSpec((D, D), lambda i: (0, 0)),
            pl.BlockSpec((1, D), lambda i: (0, 0)),
        ],
        out_specs=pl.BlockSpec((BLOCK_M, D), lambda i: (i, 0)),
        out_shape=jax.ShapeDtypeStruct((N, D), jnp.float32),
    )(x, wt, b.reshape(1, D))


def _agg_mm_body(h_ref, p0_ref, p1_ref, w_ref, b_ref, o_ref, *, relu):
    s = h_ref[...] + p0_ref[...] + p1_ref[...]
    acc = jnp.dot(s, w_ref[...], preferred_element_type=jnp.float32)
    acc = acc + b_ref[...]
    if relu:
        acc = jnp.maximum(acc, 0.0)
    o_ref[...] = acc


def _agg_mm(h, parts, wt, b, relu=False):
    grid = (N // BLOCK_M,)
    return pl.pallas_call(
        functools.partial(_agg_mm_body, relu=relu),
        grid=grid,
        in_specs=[
            pl.BlockSpec((BLOCK_M, D), lambda i: (i, 0)),
            pl.BlockSpec((BLOCK_M, D), lambda i: (i, 0)),
            pl.BlockSpec((BLOCK_M, D), lambda i: (i, 0)),
            pl.BlockSpec((D, D), lambda i: (0, 0)),
            pl.BlockSpec((1, D), lambda i: (0, 0)),
        ],
        out_specs=pl.BlockSpec((BLOCK_M, D), lambda i: (i, 0)),
        out_shape=jax.ShapeDtypeStruct((N, D), jnp.float32),
    )(h, parts[0], parts[1], wt, b.reshape(1, D))


def _sc_agg(h, src3, dst3):
    mesh = plsc.VectorSubcoreMesh(
        core_axis_name="c", subcore_axis_name="s", num_cores=NC, num_subcores=NS
    )

    @functools.partial(
        pl.kernel,
        out_type=jax.ShapeDtypeStruct((NC, N, D), jnp.float32),
        mesh=mesh,
        scratch_types=[
            pltpu.VMEM((E_TILE,), jnp.int32),
            pltpu.VMEM((N_CHUNKS, CHUNK), jnp.int32),
            pltpu.VMEM((CHUNK, D), jnp.float32),
            pltpu.VMEM((CHUNK, D), jnp.float32),
            pltpu.VMEM((ZROWS, D), jnp.float32),
            pltpu.VMEM_SHARED((N, D), jnp.float32),
            pltpu.SemaphoreType.DMA,
            pltpu.SemaphoreType.DMA,
            pltpu.SemaphoreType.DMA,
            pltpu.SemaphoreType.DMA,
        ],
    )
    def k(h_hbm, src_hbm, dst_hbm, out_hbm, src_v, dst_v, rows0, rows1,
          zeros_v, acc_sh, gsem0, gsem1, ssem0, ssem1):
        cid = lax.axis_index("c")
        sid = lax.axis_index("s")

        @pl.loop(0, ZROWS)
        def _(r):
            @pl.loop(0, D, step=16)
            def _(j):
                zeros_v[r, pl.ds(j, 16)] = jnp.zeros((16,), jnp.float32)

        row0 = sid * ROWS_PER_TILE

        @pl.loop(0, ROWS_PER_TILE, step=ZROWS)
        def _(r0):
            pltpu.sync_copy(zeros_v, acc_sh.at[pl.ds(row0 + r0, ZROWS)])

        @pl.when(sid == NS - 1)
        def _():
            pltpu.sync_copy(zeros_v, acc_sh.at[pl.ds(TAIL_ROW, TAIL)])

        plsc.subcore_barrier()

        wid = cid * NS + sid
        pltpu.sync_copy(src_hbm.at[pl.ds(wid * E_TILE, E_TILE)], src_v)
        pltpu.sync_copy(dst_hbm.at[wid], dst_v)

        def _src(j):
            return src_v.at[pl.ds(j * CHUNK, CHUNK)]

        def _gwait(j, rows, gsem):
            pltpu.make_async_copy(h_hbm.at[_src(j)], rows, gsem).wait()

        def _swait(j, rows, ssem):
            pltpu.make_async_copy(rows, acc_sh.at[dst_v.at[j]], ssem).wait()

        pltpu.async_copy(h_hbm.at[_src(0)], rows0, gsem0)

        _gwait(0, rows0, gsem0)
        pltpu.async_copy(rows0, acc_sh.at[dst_v.at[0]], ssem0, add=True)
        pltpu.async_copy(h_hbm.at[_src(1)], rows1, gsem1)

        @pl.loop(0, (N_CHUNKS - 2) // 2)
        def _(i):
            j = 2 * i + 1
            _gwait(j, rows1, gsem1)
            pltpu.async_copy(rows1, acc_sh.at[dst_v.at[j]], ssem1, add=True)
            _swait(j - 1, rows0, ssem0)
            pltpu.async_copy(h_hbm.at[_src(j + 1)], rows0, gsem0)

            _gwait(j + 1, rows0, gsem0)
            pltpu.async_copy(rows0, acc_sh.at[dst_v.at[j + 1]], ssem0, add=True)
            _swait(j, rows1, ssem1)
            pltpu.async_copy(h_hbm.at[_src(j + 2)], rows1, gsem1)

        jt = N_CHUNKS - 2
        _gwait(jt, rows1, gsem1)
        pltpu.async_copy(rows1, acc_sh.at[dst_v.at[jt]], ssem1, add=True)
        _swait(jt - 1, rows0, ssem0)
        pltpu.async_copy(h_hbm.at[_src(jt + 1)], rows0, gsem0)

        _gwait(jt + 1, rows0, gsem0)
        pltpu.async_copy(rows0, acc_sh.at[dst_v.at[jt + 1]], ssem0, add=True)
        _swait(jt, rows1, ssem1)
        _swait(jt + 1, rows0, ssem0)

        plsc.subcore_barrier()

        pltpu.sync_copy(
            acc_sh.at[pl.ds(row0, ROWS_PER_TILE)],
            out_hbm.at[cid, pl.ds(row0, ROWS_PER_TILE)],
        )

        @pl.when(sid == NS - 1)
        def _():
            pltpu.sync_copy(
                acc_sh.at[pl.ds(TAIL_ROW, TAIL)],
                out_hbm.at[cid, pl.ds(TAIL_ROW, TAIL)],
            )

    return k(h, src3, dst3)


@jax.jit
def kernel(x, edge_index, W_pre, b_pre, W1, b1, W2, b2):
    src3 = edge_index[0]
    dst3 = edge_index[1].reshape(NW, N_CHUNKS, CHUNK)
    h0 = _mm(x, W_pre.T, b_pre)
    p = _sc_agg(h0, src3, dst3)
    h1 = _agg_mm(h0, (p[0], p[1]), W1.T, b1, relu=True)
    q = _sc_agg(h1, src3, dst3)
    out = _agg_mm(h1, (q[0], q[1]), W2.T, b2)
    return out


# speedup vs baseline: 1.0002x; 1.0002x over previous
import functools

import jax
import jax.numpy as jnp
from jax import lax
from jax.experimental import pallas as pl
from jax.experimental.pallas import tpu as pltpu
from jax.experimental.pallas import tpu_sc as plsc

N = 10000
D = 128
E = 320000

NC = 2
NS = 16
NW = NC * NS

E_TILE = E // NW
CHUNK = 80
N_CHUNKS = E_TILE // CHUNK
ROWS_PER_TILE = 624
TAIL_ROW = NS * ROWS_PER_TILE
TAIL = N - TAIL_ROW
ZROWS = 16

BLOCK_M = 1000


def _fold_mm_body(x_ref, wpt_ref, w1t_ref, bp_ref, o_ref, wct_s, bc_s):
    @pl.when(pl.program_id(0) == 0)
    def _():
        wct_s[...] = jnp.dot(wpt_ref[...], w1t_ref[...],
                             preferred_element_type=jnp.float32)
        bc_s[...] = jnp.dot(bp_ref[...], w1t_ref[...],
                            preferred_element_type=jnp.float32)

    o_ref[...] = (
        jnp.dot(x_ref[...], wct_s[...], preferred_element_type=jnp.float32)
        + bc_s[...]
    )


def _fold_mm(x, wpt, w1t, bp):
    grid = (N // BLOCK_M,)
    return pl.pallas_call(
        _fold_mm_body,
        grid=grid,
        in_specs=[
            pl.BlockSpec((BLOCK_M, D), lambda i: (i, 0)),
            pl.BlockSpec((D, D), lambda i: (0, 0)),
            pl.BlockSpec((D, D), lambda i: (0, 0)),
            pl.BlockSpec((1, D), lambda i: (0, 0)),
        ],
        out_specs=pl.BlockSpec((BLOCK_M, D), lambda i: (i, 0)),
        out_shape=jax.ShapeDtypeStruct((N, D), jnp.float32),
        scratch_shapes=[
            pltpu.VMEM((D, D), jnp.float32),
            pltpu.VMEM((1, D), jnp.float32),
        ],
    )(x, wpt, w1t, bp.reshape(1, D))


def _agg_relu_mm_body(h_ref, p0_ref, p1_ref, w_ref, b_ref, o_ref):
    s = jnp.maximum(h_ref[...] + p0_ref[...] + p1_ref[...] + b_ref[...], 0.0)
    o_ref[...] = jnp.dot(s, w_ref[...], preferred_element_type=jnp.float32)


def _agg_relu_mm(h, parts, b1, w2t):
    grid = (N // BLOCK_M,)
    return pl.pallas_call(
        _agg_relu_mm_body,
        grid=grid,
        in_specs=[
            pl.BlockSpec((BLOCK_M, D), lambda i: (i, 0)),
            pl.BlockSpec((BLOCK_M, D), lambda i: (i, 0)),
            pl.BlockSpec((BLOCK_M, D), lambda i: (i, 0)),
            pl.BlockSpec((D, D), lambda i: (0, 0)),
            pl.BlockSpec((1, D), lambda i: (0, 0)),
        ],
        out_specs=pl.BlockSpec((BLOCK_M, D), lambda i: (i, 0)),
        out_shape=jax.ShapeDtypeStruct((N, D), jnp.float32),
    )(h, parts[0], parts[1], w2t, b1.reshape(1, D))


def _final_add_body(g_ref, q0_ref, q1_ref, b_ref, o_ref):
    o_ref[...] = g_ref[...] + q0_ref[...] + q1_ref[...] + b_ref[...]


def _final_add(g, parts, b2):
    grid = (N // BLOCK_M,)
    return pl.pallas_call(
        _final_add_body,
        grid=grid,
        in_specs=[
            pl.BlockSpec((BLOCK_M, D), lambda i: (i, 0)),
            pl.BlockSpec((BLOCK_M, D), lambda i: (i, 0)),
            pl.BlockSpec((BLOCK_M, D), lambda i: (i, 0)),
            pl.BlockSpec((1, D), lambda i: (0, 0)),
        ],
        out_specs=pl.BlockSpec((BLOCK_M, D), lambda i: (i, 0)),
        out_shape=jax.ShapeDtypeStruct((N, D), jnp.float32),
    )(g, parts[0], parts[1], b2.reshape(1, D))


def _sc_agg(h, src3, dst3):
    mesh = plsc.VectorSubcoreMesh(
        core_axis_name="c", subcore_axis_name="s", num_cores=NC, num_subcores=NS
    )

    @functools.partial(
        pl.kernel,
        out_type=jax.ShapeDtypeStruct((NC, N, D), jnp.float32),
        mesh=mesh,
        scratch_types=[
            pltpu.VMEM((E_TILE,), jnp.int32),
            pltpu.VMEM((N_CHUNKS, CHUNK), jnp.int32),
            pltpu.VMEM((CHUNK, D), jnp.float32),
            pltpu.VMEM((CHUNK, D), jnp.float32),
            pltpu.VMEM((ZROWS, D), jnp.float32),
            pltpu.VMEM_SHARED((N, D), jnp.float32),
            pltpu.SemaphoreType.DMA,
            pltpu.SemaphoreType.DMA,
            pltpu.SemaphoreType.DMA,
            pltpu.SemaphoreType.DMA,
        ],
    )
    def k(h_hbm, src_hbm, dst_hbm, out_hbm, src_v, dst_v, rows0, rows1,
          zeros_v, acc_sh, gsem0, gsem1, ssem0, ssem1):
        cid = lax.axis_index("c")
        sid = lax.axis_index("s")

        @pl.loop(0, ZROWS)
        def _(r):
            @pl.loop(0, D, step=16)
            def _(j):
                zeros_v[r, pl.ds(j, 16)] = jnp.zeros((16,), jnp.float32)

        row0 = sid * ROWS_PER_TILE

        @pl.loop(0, ROWS_PER_TILE, step=ZROWS)
        def _(r0):
            pltpu.sync_copy(zeros_v, acc_sh.at[pl.ds(row0 + r0, ZROWS)])

        @pl.when(sid == NS - 1)
        def _():
            pltpu.sync_copy(zeros_v, acc_sh.at[pl.ds(TAIL_ROW, TAIL)])

        plsc.subcore_barrier()

        wid = cid * NS + sid
        pltpu.sync_copy(src_hbm.at[pl.ds(wid * E_TILE, E_TILE)], src_v)
        pltpu.sync_copy(dst_hbm.at[wid], dst_v)

        def _src(j):
            return src_v.at[pl.ds(j * CHUNK, CHUNK)]

        def _gwait(j, rows, gsem):
            pltpu.make_async_copy(h_hbm.at[_src(j)], rows, gsem).wait()

        def _swait(j, rows, ssem):
            pltpu.make_async_copy(rows, acc_sh.at[dst_v.at[j]], ssem).wait()

        pltpu.async_copy(h_hbm.at[_src(0)], rows0, gsem0)

        _gwait(0, rows0, gsem0)
        pltpu.async_copy(rows0, acc_sh.at[dst_v.at[0]], ssem0, add=True)
        pltpu.async_copy(h_hbm.at[_src(1)], rows1, gsem1)

        @pl.loop(0, (N_CHUNKS - 2) // 2)
        def _(i):
            j = 2 * i + 1
            _gwait(j, rows1, gsem1)
            pltpu.async_copy(rows1, acc_sh.at[dst_v.at[j]], ssem1, add=True)
            _swait(j - 1, rows0, ssem0)
            pltpu.async_copy(h_hbm.at[_src(j + 1)], rows0, gsem0)

            _gwait(j + 1, rows0, gsem0)
            pltpu.async_copy(rows0, acc_sh.at[dst_v.at[j + 1]], ssem0, add=True)
            _swait(j, rows1, ssem1)
            pltpu.async_copy(h_hbm.at[_src(j + 2)], rows1, gsem1)

        jt = N_CHUNKS - 2
        _gwait(jt, rows1, gsem1)
        pltpu.async_copy(rows1, acc_sh.at[dst_v.at[jt]], ssem1, add=True)
        _swait(jt - 1, rows0, ssem0)
        pltpu.async_copy(h_hbm.at[_src(jt + 1)], rows0, gsem0)

        _gwait(jt + 1, rows0, gsem0)
        pltpu.async_copy(rows0, acc_sh.at[dst_v.at[jt + 1]], ssem0, add=True)
        _swait(jt, rows1, ssem1)
        _swait(jt + 1, rows0, ssem0)

        plsc.subcore_barrier()

        pltpu.sync_copy(
            acc_sh.at[pl.ds(row0, ROWS_PER_TILE)],
            out_hbm.at[cid, pl.ds(row0, ROWS_PER_TILE)],
        )

        @pl.when(sid == NS - 1)
        def _():
            pltpu.sync_copy(
                acc_sh.at[pl.ds(TAIL_ROW, TAIL)],
                out_hbm.at[cid, pl.ds(TAIL_ROW, TAIL)],
            )

    return k(h, src3, dst3)


@jax.jit
def kernel(x, edge_index, W_pre, b_pre, W1, b1, W2, b2):
    src = edge_index[0]
    dst3 = edge_index[1].reshape(NW, N_CHUNKS, CHUNK)
    g1 = _fold_mm(x, W_pre.T, W1.T, b_pre)
    p = _sc_agg(g1, src, dst3)
    g2 = _agg_relu_mm(g1, (p[0], p[1]), b1, W2.T)
    q = _sc_agg(g2, src, dst3)
    return _final_add(g2, (q[0], q[1]), b2)


# speedup vs baseline: 1.0458x; 1.0455x over previous
import functools

import jax
import jax.numpy as jnp
from jax import lax
from jax.experimental import pallas as pl
from jax.experimental.pallas import tpu as pltpu
from jax.experimental.pallas import tpu_sc as plsc

N = 10000
D = 128
E = 320000

NC = 2
NS = 16
NW = NC * NS

E_TILE = E // NW
CHUNK = 80
N_CHUNKS = E_TILE // CHUNK
ROWS_PER_TILE = 624
TAIL_ROW = NS * ROWS_PER_TILE
TAIL = N - TAIL_ROW
ZROWS = 16

BLOCK_M = 2000


def _fold_mm_body(x_ref, wpt_ref, w1t_ref, bp_ref, o_ref, wct_s, bc_s):
    @pl.when(pl.program_id(0) == 0)
    def _():
        wct_s[...] = jnp.dot(wpt_ref[...], w1t_ref[...],
                             preferred_element_type=jnp.float32)
        bc_s[...] = jnp.dot(bp_ref[...], w1t_ref[...],
                            preferred_element_type=jnp.float32)

    o_ref[...] = (
        jnp.dot(x_ref[...], wct_s[...], preferred_element_type=jnp.float32)
        + bc_s[...]
    )


def _fold_mm(x, wpt, w1t, bp):
    grid = (N // BLOCK_M,)
    return pl.pallas_call(
        _fold_mm_body,
        grid=grid,
        in_specs=[
            pl.BlockSpec((BLOCK_M, D), lambda i: (i, 0)),
            pl.BlockSpec((D, D), lambda i: (0, 0)),
            pl.BlockSpec((D, D), lambda i: (0, 0)),
            pl.BlockSpec((1, D), lambda i: (0, 0)),
        ],
        out_specs=pl.BlockSpec((BLOCK_M, D), lambda i: (i, 0)),
        out_shape=jax.ShapeDtypeStruct((N, D), jnp.float32),
        scratch_shapes=[
            pltpu.VMEM((D, D), jnp.float32),
            pltpu.VMEM((1, D), jnp.float32),
        ],
    )(x, wpt, w1t, bp.reshape(1, D))


def _agg_relu_mm_body(h_ref, p0_ref, p1_ref, w_ref, b_ref, o_ref):
    s = jnp.maximum(h_ref[...] + p0_ref[...] + p1_ref[...] + b_ref[...], 0.0)
    o_ref[...] = jnp.dot(s, w_ref[...], preferred_element_type=jnp.float32)


def _agg_relu_mm(h, parts, b1, w2t):
    grid = (N // BLOCK_M,)
    return pl.pallas_call(
        _agg_relu_mm_body,
        grid=grid,
        in_specs=[
            pl.BlockSpec((BLOCK_M, D), lambda i: (i, 0)),
            pl.BlockSpec((BLOCK_M, D), lambda i: (i, 0)),
            pl.BlockSpec((BLOCK_M, D), lambda i: (i, 0)),
            pl.BlockSpec((D, D), lambda i: (0, 0)),
            pl.BlockSpec((1, D), lambda i: (0, 0)),
        ],
        out_specs=pl.BlockSpec((BLOCK_M, D), lambda i: (i, 0)),
        out_shape=jax.ShapeDtypeStruct((N, D), jnp.float32),
    )(h, parts[0], parts[1], w2t, b1.reshape(1, D))


def _final_add_body(g_ref, q0_ref, q1_ref, b_ref, o_ref):
    o_ref[...] = g_ref[...] + q0_ref[...] + q1_ref[...] + b_ref[...]


def _final_add(g, parts, b2):
    grid = (N // BLOCK_M,)
    return pl.pallas_call(
        _final_add_body,
        grid=grid,
        in_specs=[
            pl.BlockSpec((BLOCK_M, D), lambda i: (i, 0)),
            pl.BlockSpec((BLOCK_M, D), lambda i: (i, 0)),
            pl.BlockSpec((BLOCK_M, D), lambda i: (i, 0)),
            pl.BlockSpec((1, D), lambda i: (0, 0)),
        ],
        out_specs=pl.BlockSpec((BLOCK_M, D), lambda i: (i, 0)),
        out_shape=jax.ShapeDtypeStruct((N, D), jnp.float32),
    )(g, parts[0], parts[1], b2.reshape(1, D))


def _sc_agg(h, src3, dst3):
    mesh = plsc.VectorSubcoreMesh(
        core_axis_name="c", subcore_axis_name="s", num_cores=NC, num_subcores=NS
    )

    @functools.partial(
        pl.kernel,
        out_type=(jax.ShapeDtypeStruct((N, D), jnp.float32),
                  jax.ShapeDtypeStruct((N, D), jnp.float32)),
        mesh=mesh,
        scratch_types=[
            pltpu.VMEM((E_TILE,), jnp.int32),
            pltpu.VMEM((N_CHUNKS, CHUNK), jnp.int32),
            pltpu.VMEM((CHUNK, D), jnp.float32),
            pltpu.VMEM((CHUNK, D), jnp.float32),
            pltpu.VMEM((ZROWS, D), jnp.float32),
            pltpu.VMEM_SHARED((N, D), jnp.float32),
            pltpu.SemaphoreType.DMA,
            pltpu.SemaphoreType.DMA,
            pltpu.SemaphoreType.DMA,
            pltpu.SemaphoreType.DMA,
        ],
    )
    def k(h_hbm, src_hbm, dst_hbm, out0_hbm, out1_hbm, src_v, dst_v, rows0,
          rows1, zeros_v, acc_sh, gsem0, gsem1, ssem0, ssem1):
        cid = lax.axis_index("c")
        sid = lax.axis_index("s")

        @pl.loop(0, ZROWS)
        def _(r):
            @pl.loop(0, D, step=16)
            def _(j):
                zeros_v[r, pl.ds(j, 16)] = jnp.zeros((16,), jnp.float32)

        row0 = sid * ROWS_PER_TILE

        @pl.loop(0, ROWS_PER_TILE, step=ZROWS)
        def _(r0):
            pltpu.sync_copy(zeros_v, acc_sh.at[pl.ds(row0 + r0, ZROWS)])

        @pl.when(sid == NS - 1)
        def _():
            pltpu.sync_copy(zeros_v, acc_sh.at[pl.ds(TAIL_ROW, TAIL)])

        plsc.subcore_barrier()

        wid = cid * NS + sid
        pltpu.sync_copy(src_hbm.at[pl.ds(wid * E_TILE, E_TILE)], src_v)
        pltpu.sync_copy(dst_hbm.at[wid], dst_v)

        def _src(j):
            return src_v.at[pl.ds(j * CHUNK, CHUNK)]

        def _gwait(j, rows, gsem):
            pltpu.make_async_copy(h_hbm.at[_src(j)], rows, gsem).wait()

        def _swait(j, rows, ssem):
            pltpu.make_async_copy(rows, acc_sh.at[dst_v.at[j]], ssem).wait()

        pltpu.async_copy(h_hbm.at[_src(0)], rows0, gsem0)

        _gwait(0, rows0, gsem0)
        pltpu.async_copy(rows0, acc_sh.at[dst_v.at[0]], ssem0, add=True)
        pltpu.async_copy(h_hbm.at[_src(1)], rows1, gsem1)

        @pl.loop(0, (N_CHUNKS - 2) // 2)
        def _(i):
            j = 2 * i + 1
            _gwait(j, rows1, gsem1)
            pltpu.async_copy(rows1, acc_sh.at[dst_v.at[j]], ssem1, add=True)
            _swait(j - 1, rows0, ssem0)
            pltpu.async_copy(h_hbm.at[_src(j + 1)], rows0, gsem0)

            _gwait(j + 1, rows0, gsem0)
            pltpu.async_copy(rows0, acc_sh.at[dst_v.at[j + 1]], ssem0, add=True)
            _swait(j, rows1, ssem1)
            pltpu.async_copy(h_hbm.at[_src(j + 2)], rows1, gsem1)

        jt = N_CHUNKS - 2
        _gwait(jt, rows1, gsem1)
        pltpu.async_copy(rows1, acc_sh.at[dst_v.at[jt]], ssem1, add=True)
        _swait(jt - 1, rows0, ssem0)
        pltpu.async_copy(h_hbm.at[_src(jt + 1)], rows0, gsem0)

        _gwait(jt + 1, rows0, gsem0)
        pltpu.async_copy(rows0, acc_sh.at[dst_v.at[jt + 1]], ssem0, add=True)
        _swait(jt, rows1, ssem1)
        _swait(jt + 1, rows0, ssem0)

        plsc.subcore_barrier()

        @pl.when(cid == 0)
        def _():
            pltpu.sync_copy(
                acc_sh.at[pl.ds(row0, ROWS_PER_TILE)],
                out0_hbm.at[pl.ds(row0, ROWS_PER_TILE)],
            )

            @pl.when(sid == NS - 1)
            def _():
                pltpu.sync_copy(
                    acc_sh.at[pl.ds(TAIL_ROW, TAIL)],
                    out0_hbm.at[pl.ds(TAIL_ROW, TAIL)],
                )

        @pl.when(cid == 1)
        def _():
            pltpu.sync_copy(
                acc_sh.at[pl.ds(row0, ROWS_PER_TILE)],
                out1_hbm.at[pl.ds(row0, ROWS_PER_TILE)],
            )

            @pl.when(sid == NS - 1)
            def _():
                pltpu.sync_copy(
                    acc_sh.at[pl.ds(TAIL_ROW, TAIL)],
                    out1_hbm.at[pl.ds(TAIL_ROW, TAIL)],
                )

    return k(h, src3, dst3)


@jax.jit
def kernel(x, edge_index, W_pre, b_pre, W1, b1, W2, b2):
    e_flat = edge_index.reshape(2 * E)
    src = e_flat[:E]
    dst3 = e_flat[E:].reshape(NW, N_CHUNKS, CHUNK)
    g1 = _fold_mm(x, W_pre.T, W1.T, b_pre)
    p = _sc_agg(g1, src, dst3)
    g2 = _agg_relu_mm(g1, (p[0], p[1]), b1, W2.T)
    q = _sc_agg(g2, src, dst3)
    return _final_add(g2, (q[0], q[1]), b2)


# speedup vs baseline: 1.3015x; 1.2445x over previous
import functools

import jax
import jax.numpy as jnp
from jax import lax
from jax.experimental import pallas as pl
from jax.experimental.pallas import tpu as pltpu
from jax.experimental.pallas import tpu_sc as plsc

N = 10000
D = 128
E = 320000

NC = 2
NS = 16
NW = NC * NS

CHUNK = 128
CHUNKS_TOTAL = E // CHUNK
TILE_CHUNKS = CHUNKS_TOTAL // NW
EXTRA_BASE = NW * TILE_CHUNKS
ROWS_PER_TILE = 624
TAIL_ROW = NS * ROWS_PER_TILE
TAIL = N - TAIL_ROW
ZROWS = 16

BLOCK_M = 2000


def _fold_mm_body(x_ref, wpt_ref, w1t_ref, bp_ref, o_ref, wct_s, bc_s):
    @pl.when(pl.program_id(0) == 0)
    def _():
        wct_s[...] = jnp.dot(wpt_ref[...], w1t_ref[...],
                             preferred_element_type=jnp.float32)
        bc_s[...] = jnp.dot(bp_ref[...], w1t_ref[...],
                            preferred_element_type=jnp.float32)

    o_ref[...] = (
        jnp.dot(x_ref[...], wct_s[...], preferred_element_type=jnp.float32)
        + bc_s[...]
    )


def _fold_mm(x, wpt, w1t, bp):
    grid = (N // BLOCK_M,)
    return pl.pallas_call(
        _fold_mm_body,
        grid=grid,
        in_specs=[
            pl.BlockSpec((BLOCK_M, D), lambda i: (i, 0)),
            pl.BlockSpec((D, D), lambda i: (0, 0)),
            pl.BlockSpec((D, D), lambda i: (0, 0)),
            pl.BlockSpec((1, D), lambda i: (0, 0)),
        ],
        out_specs=pl.BlockSpec((BLOCK_M, D), lambda i: (i, 0)),
        out_shape=jax.ShapeDtypeStruct((N, D), jnp.float32),
        scratch_shapes=[
            pltpu.VMEM((D, D), jnp.float32),
            pltpu.VMEM((1, D), jnp.float32),
        ],
    )(x, wpt, w1t, bp.reshape(1, D))


def _agg_relu_mm_body(h_ref, p0_ref, p1_ref, w_ref, b_ref, o_ref):
    s = jnp.maximum(h_ref[...] + p0_ref[...] + p1_ref[...] + b_ref[...], 0.0)
    o_ref[...] = jnp.dot(s, w_ref[...], preferred_element_type=jnp.float32)


def _agg_relu_mm(h, parts, b1, w2t):
    grid = (N // BLOCK_M,)
    return pl.pallas_call(
        _agg_relu_mm_body,
        grid=grid,
        in_specs=[
            pl.BlockSpec((BLOCK_M, D), lambda i: (i, 0)),
            pl.BlockSpec((BLOCK_M, D), lambda i: (i, 0)),
            pl.BlockSpec((BLOCK_M, D), lambda i: (i, 0)),
            pl.BlockSpec((D, D), lambda i: (0, 0)),
            pl.BlockSpec((1, D), lambda i: (0, 0)),
        ],
        out_specs=pl.BlockSpec((BLOCK_M, D), lambda i: (i, 0)),
        out_shape=jax.ShapeDtypeStruct((N, D), jnp.float32),
    )(h, parts[0], parts[1], w2t, b1.reshape(1, D))


def _final_add_body(g_ref, q0_ref, q1_ref, b_ref, o_ref):
    o_ref[...] = g_ref[...] + q0_ref[...] + q1_ref[...] + b_ref[...]


def _final_add(g, parts, b2):
    grid = (N // BLOCK_M,)
    return pl.pallas_call(
        _final_add_body,
        grid=grid,
        in_specs=[
            pl.BlockSpec((BLOCK_M, D), lambda i: (i, 0)),
            pl.BlockSpec((BLOCK_M, D), lambda i: (i, 0)),
            pl.BlockSpec((BLOCK_M, D), lambda i: (i, 0)),
            pl.BlockSpec((1, D), lambda i: (0, 0)),
        ],
        out_specs=pl.BlockSpec((BLOCK_M, D), lambda i: (i, 0)),
        out_shape=jax.ShapeDtypeStruct((N, D), jnp.float32),
    )(g, parts[0], parts[1], b2.reshape(1, D))


def _sc_agg(h, edge_index):
    mesh = plsc.VectorSubcoreMesh(
        core_axis_name="c", subcore_axis_name="s", num_cores=NC, num_subcores=NS
    )

    @functools.partial(
        pl.kernel,
        out_type=(jax.ShapeDtypeStruct((N, D), jnp.float32),
                  jax.ShapeDtypeStruct((N, D), jnp.float32)),
        mesh=mesh,
        scratch_types=[
            [pltpu.VMEM((2, CHUNK), jnp.int32) for _ in range(4)],
            [pltpu.VMEM((CHUNK, D), jnp.float32) for _ in range(2)],
            pltpu.VMEM((ZROWS, D), jnp.float32),
            pltpu.VMEM_SHARED((N, D), jnp.float32),
            [pltpu.SemaphoreType.DMA for _ in range(4)],
            [pltpu.SemaphoreType.DMA for _ in range(2)],
            [pltpu.SemaphoreType.DMA for _ in range(2)],
        ],
    )
    def k(h_hbm, e_hbm, out0_hbm, out1_hbm, ib, rows, zeros_v, acc_sh,
          isem, gsem, ssem):
        cid = lax.axis_index("c")
        sid = lax.axis_index("s")

        @pl.loop(0, ZROWS)
        def _(r):
            @pl.loop(0, D, step=16)
            def _(j):
                zeros_v[r, pl.ds(j, 16)] = jnp.zeros((16,), jnp.float32)

        row0 = sid * ROWS_PER_TILE

        @pl.loop(0, ROWS_PER_TILE, step=ZROWS)
        def _(r0):
            pltpu.sync_copy(zeros_v, acc_sh.at[pl.ds(row0 + r0, ZROWS)])

        @pl.when(sid == NS - 1)
        def _():
            pltpu.sync_copy(zeros_v, acc_sh.at[pl.ds(TAIL_ROW, TAIL)])

        plsc.subcore_barrier()

        wid = cid * NS + sid
        base = wid * TILE_CHUNKS

        def _echunk(j):
            return e_hbm.at[pl.ds(0, 2), pl.ds((base + j) * CHUNK, CHUNK)]

        def _iload(j, m4):
            pltpu.async_copy(_echunk(j), ib[m4], isem[m4])

        def _iwait(j, m4):
            pltpu.make_async_copy(_echunk(j), ib[m4], isem[m4]).wait()

        def _gstart(j, m4, m2):
            pltpu.async_copy(h_hbm.at[ib[m4].at[0]], rows[m2], gsem[m2])

        def _gwait(j, m4, m2):
            pltpu.make_async_copy(h_hbm.at[ib[m4].at[0]], rows[m2],
                                  gsem[m2]).wait()

        def _sstart(j, m4, m2):
            pltpu.async_copy(rows[m2], acc_sh.at[ib[m4].at[1]], ssem[m2],
                             add=True)

        def _swait(j, m4, m2):
            pltpu.make_async_copy(rows[m2], acc_sh.at[ib[m4].at[1]],
                                  ssem[m2]).wait()

        def _step(j, jm, first=False, do_next=True, do_pref=True):
            m4, m2 = jm % 4, jm % 2
            n4, n2 = (jm + 1) % 4, (jm + 1) % 2
            _gwait(j, m4, m2)
            _sstart(j, m4, m2)
            if not first:
                _swait(j - 1, (jm + 3) % 4, n2)
            if do_next:
                _iwait(j + 1, n4)
                _gstart(j + 1, n4, n2)
            if do_pref:
                _iload(j + 3, (jm + 3) % 4)

        _iload(0, 0)
        _iload(1, 1)
        _iload(2, 2)
        _iwait(0, 0)
        _gstart(0, 0, 0)

        _step(0, 0, first=True)
        _step(1, 1)
        _step(2, 2)
        _step(3, 3)

        @pl.loop(0, (TILE_CHUNKS - 10) // 4)
        def _(g):
            j = 4 * g + 4
            _step(j + 0, 0)
            _step(j + 1, 1)
            _step(j + 2, 2)
            _step(j + 3, 3)

        _step(TILE_CHUNKS - 6, (TILE_CHUNKS - 6) % 4)
        _step(TILE_CHUNKS - 5, (TILE_CHUNKS - 5) % 4)
        _step(TILE_CHUNKS - 4, (TILE_CHUNKS - 4) % 4)
        _step(TILE_CHUNKS - 3, (TILE_CHUNKS - 3) % 4, do_pref=False)
        _step(TILE_CHUNKS - 2, (TILE_CHUNKS - 2) % 4, do_pref=False)
        _step(TILE_CHUNKS - 1, (TILE_CHUNKS - 1) % 4, do_next=False,
              do_pref=False)
        _swait(TILE_CHUNKS - 1, (TILE_CHUNKS - 1) % 4, (TILE_CHUNKS - 1) % 2)

        @pl.when(wid < CHUNKS_TOTAL - EXTRA_BASE)
        def _():
            ec = e_hbm.at[pl.ds(0, 2),
                          pl.ds((EXTRA_BASE + wid) * CHUNK, CHUNK)]
            pltpu.sync_copy(ec, ib[0])
            pltpu.async_copy(h_hbm.at[ib[0].at[0]], rows[0], gsem[0]).wait()
            pltpu.sync_copy(rows[0], acc_sh.at[ib[0].at[1]], add=True)

        plsc.subcore_barrier()

        @pl.when(cid == 0)
        def _():
            pltpu.sync_copy(
                acc_sh.at[pl.ds(row0, ROWS_PER_TILE)],
                out0_hbm.at[pl.ds(row0, ROWS_PER_TILE)],
            )

            @pl.when(sid == NS - 1)
            def _():
                pltpu.sync_copy(
                    acc_sh.at[pl.ds(TAIL_ROW, TAIL)],
                    out0_hbm.at[pl.ds(TAIL_ROW, TAIL)],
                )

        @pl.when(cid == 1)
        def _():
            pltpu.sync_copy(
                acc_sh.at[pl.ds(row0, ROWS_PER_TILE)],
                out1_hbm.at[pl.ds(row0, ROWS_PER_TILE)],
            )

            @pl.when(sid == NS - 1)
            def _():
                pltpu.sync_copy(
                    acc_sh.at[pl.ds(TAIL_ROW, TAIL)],
                    out1_hbm.at[pl.ds(TAIL_ROW, TAIL)],
                )

    return k(h, edge_index)


@jax.jit
def kernel(x, edge_index, W_pre, b_pre, W1, b1, W2, b2):
    g1 = _fold_mm(x, W_pre.T, W1.T, b_pre)
    p = _sc_agg(g1, edge_index)
    g2 = _agg_relu_mm(g1, (p[0], p[1]), b1, W2.T)
    q = _sc_agg(g2, edge_index)
    return _final_add(g2, (q[0], q[1]), b2)
